# baseline (speedup 1.0000x reference)
import functools

import jax
import jax.numpy as jnp
import numpy as np
from jax import lax
from jax.experimental import pallas as pl
from jax.experimental.pallas import tpu as pltpu
from jax.experimental.pallas import tpu_sc as plsc

_DIM_USED = np.array(
    [6, 7, 8, 9, 10, 11, 12, 13, 14, 15, 16, 17, 21, 22, 23, 24, 25, 26,
     27, 28, 29, 30, 31, 32, 36, 37, 38, 39, 40, 41, 42, 43, 44, 45, 46,
     47, 51, 52, 53, 54, 55, 56, 57, 58, 59, 63, 64, 65, 66, 67, 68, 75,
     76, 77, 78, 79, 80, 81, 82, 83, 87, 88, 89, 90, 91, 92],
    dtype=np.int32,
)

D_IN = 96
D_OUT = 66
NBATCH = 4096
SEQ = 50
NC, NS = 1, 16
NW = NC * NS
BATCH_PER_TILE = NBATCH // NW
CHUNK_B = 4
NCHUNK = BATCH_PER_TILE // CHUNK_B
CHUNK_ROWS = CHUNK_B * SEQ
BLOCKS = CHUNK_ROWS // 8
NVEC = 8 * D_OUT // 16
NPAIR = NCHUNK // 2

_OFFS = (0, 16, 32, 48, 50)
_TAB = np.stack([_DIM_USED[o:o + 16] for o in _OFFS]).astype(np.int32)


@functools.partial(
    pl.kernel,
    out_type=jax.ShapeDtypeStruct((NBATCH, SEQ, D_OUT), jnp.float32),
    mesh=plsc.VectorSubcoreMesh(core_axis_name="c", subcore_axis_name="s",
                                num_cores=NC),
    scratch_types=[
        pltpu.VMEM((len(_OFFS), 16), jnp.int32),
        pltpu.VMEM((CHUNK_ROWS, D_IN), jnp.float32),
        pltpu.VMEM((CHUNK_ROWS, D_IN), jnp.float32),
        pltpu.VMEM((CHUNK_ROWS, D_OUT), jnp.float32),
        pltpu.VMEM((CHUNK_ROWS, D_OUT), jnp.float32),
        pltpu.SemaphoreType.DMA,
        pltpu.SemaphoreType.DMA,
        pltpu.SemaphoreType.DMA,
        pltpu.SemaphoreType.DMA,
    ],
    compiler_params=pltpu.CompilerParams(needs_layout_passes=False),
)
def _sc_select(x_hbm, tab_hbm, out_hbm, tab_v, in_v0, in_v1, out_v0, out_v1,
               si0, si1, so0, so1):
    wid = lax.axis_index("s") * NC + lax.axis_index("c")
    pltpu.sync_copy(tab_hbm, tab_v)
    base = wid * BATCH_PER_TILE
    lanes = lax.iota(jnp.int32, 16)
    zero16 = jnp.zeros((16,), jnp.int32)
    in_v = (in_v0, in_v1)
    out_v = (out_v0, out_v1)
    sin = (si0, si1)
    sout = (so0, so1)

    def start_in(c, p):
        b0 = base + c * CHUNK_B
        for i in range(CHUNK_B):
            pltpu.async_copy(
                x_hbm.at[b0 + i], in_v[p].at[pl.ds(SEQ * i, SEQ), :], sin[p])

    def wait_in(c, p):
        b0 = base + c * CHUNK_B
        for i in range(CHUNK_B):
            pltpu.make_async_copy(
                x_hbm.at[b0 + i], in_v[p].at[pl.ds(SEQ * i, SEQ), :],
                sin[p]).wait()

    def start_out(c, p):
        b0 = base + c * CHUNK_B
        for i in range(CHUNK_B):
            pltpu.async_copy(
                out_v[p].at[pl.ds(SEQ * i, SEQ), :], out_hbm.at[b0 + i],
                sout[p])

    def wait_out(c, p):
        b0 = base + c * CHUNK_B
        for i in range(CHUNK_B):
            pltpu.make_async_copy(
                out_v[p].at[pl.ds(SEQ * i, SEQ), :], out_hbm.at[b0 + i],
                sout[p]).wait()

    colvs = tuple(tab_v[k, :] for k in range(len(_OFFS)))
    UNROLL = 10

    def repack(p):
        src = in_v[p]
        dst = out_v[p]

        def rbody(t, carry):
            r0 = t * UNROLL
            for u in range(UNROLL):
                rv = jnp.full((16,), r0 + u, jnp.int32)
                for off, cv in zip(_OFFS, colvs):
                    dst[r0 + u, pl.ds(off, 16)] = plsc.load_gather(
                        src, [rv, cv])
            return carry

        lax.fori_loop(0, CHUNK_ROWS // UNROLL, rbody, 0)

    start_in(0, 0)

    def pair_body(t, carry):
        c0 = 2 * t

        start_in(c0 + 1, 1)
        wait_in(c0, 0)

        @pl.when(t >= 1)
        def _():
            wait_out(c0 - 2, 0)

        repack(0)
        start_out(c0, 0)

        @pl.when(t + 1 < NPAIR)
        def _():
            start_in(c0 + 2, 0)

        wait_in(c0 + 1, 1)

        @pl.when(t >= 1)
        def _():
            wait_out(c0 - 1, 1)

        repack(1)
        start_out(c0 + 1, 1)
        return carry

    lax.fori_loop(0, NPAIR, pair_body, 0)
    wait_out(NCHUNK - 2, 0)
    wait_out(NCHUNK - 1, 1)


def kernel(observed_pose):
    return _sc_select(observed_pose, jnp.asarray(_TAB))


# speedup vs baseline: 1.3623x; 1.3623x over previous
import functools

import jax
import jax.numpy as jnp
import numpy as np
from jax import lax
from jax.experimental import pallas as pl
from jax.experimental.pallas import tpu as pltpu
from jax.experimental.pallas import tpu_sc as plsc

_DIM_USED = np.array(
    [6, 7, 8, 9, 10, 11, 12, 13, 14, 15, 16, 17, 21, 22, 23, 24, 25, 26,
     27, 28, 29, 30, 31, 32, 36, 37, 38, 39, 40, 41, 42, 43, 44, 45, 46,
     47, 51, 52, 53, 54, 55, 56, 57, 58, 59, 63, 64, 65, 66, 67, 68, 75,
     76, 77, 78, 79, 80, 81, 82, 83, 87, 88, 89, 90, 91, 92],
    dtype=np.int32,
)

D_IN = 96
D_OUT = 66
NBATCH = 4096
SEQ = 50
NC, NS = 2, 16
NW = NC * NS
BATCH_PER_TILE = NBATCH // NW
CHUNK_B = 4
NCHUNK = BATCH_PER_TILE // CHUNK_B
CHUNK_ROWS = CHUNK_B * SEQ
BLOCKS = CHUNK_ROWS // 8
NVEC = 8 * D_OUT // 16
NPAIR = NCHUNK // 2

_OFFS = (0, 16, 32, 48, 50)
_TAB = np.stack([_DIM_USED[o:o + 16] for o in _OFFS]).astype(np.int32)


@functools.partial(
    pl.kernel,
    out_type=jax.ShapeDtypeStruct((NBATCH, SEQ, D_OUT), jnp.float32),
    mesh=plsc.VectorSubcoreMesh(core_axis_name="c", subcore_axis_name="s",
                                num_cores=NC),
    scratch_types=[
        pltpu.VMEM((CHUNK_ROWS, D_IN), jnp.float32),
        pltpu.VMEM((CHUNK_ROWS, D_IN), jnp.float32),
        pltpu.VMEM((CHUNK_ROWS, D_OUT), jnp.float32),
        pltpu.VMEM((CHUNK_ROWS, D_OUT), jnp.float32),
        pltpu.SemaphoreType.DMA,
        pltpu.SemaphoreType.DMA,
        pltpu.SemaphoreType.DMA,
        pltpu.SemaphoreType.DMA,
    ],
    compiler_params=pltpu.CompilerParams(needs_layout_passes=False),
)
def _sc_select(x_hbm, out_hbm, in_v0, in_v1, out_v0, out_v1,
               si0, si1, so0, so1):
    wid = lax.axis_index("s") * NC + lax.axis_index("c")
    base = wid * BATCH_PER_TILE
    lanes = lax.iota(jnp.int32, 16)
    zero16 = jnp.zeros((16,), jnp.int32)
    in_v = (in_v0, in_v1)
    out_v = (out_v0, out_v1)
    sin = (si0, si1)
    sout = (so0, so1)

    def start_in(c, p):
        b0 = base + c * CHUNK_B
        for i in range(CHUNK_B):
            pltpu.async_copy(
                x_hbm.at[b0 + i], in_v[p].at[pl.ds(SEQ * i, SEQ), :], sin[p])

    def wait_in(c, p):
        b0 = base + c * CHUNK_B
        for i in range(CHUNK_B):
            pltpu.make_async_copy(
                x_hbm.at[b0 + i], in_v[p].at[pl.ds(SEQ * i, SEQ), :],
                sin[p]).wait()

    def start_out(c, p):
        b0 = base + c * CHUNK_B
        for i in range(CHUNK_B):
            pltpu.async_copy(
                out_v[p].at[pl.ds(SEQ * i, SEQ), :], out_hbm.at[b0 + i],
                sout[p])

    def wait_out(c, p):
        b0 = base + c * CHUNK_B
        for i in range(CHUNK_B):
            pltpu.make_async_copy(
                out_v[p].at[pl.ds(SEQ * i, SEQ), :], out_hbm.at[b0 + i],
                sout[p]).wait()

    def col_vec(off):
        vals = _DIM_USED[off:off + 16]
        c = lanes + int(vals[0])
        for l in range(1, 16):
            jump = int(vals[l]) - int(vals[l - 1]) - 1
            if jump:
                c = c + jnp.where(lanes >= l, jump, 0)
        return c

    colvs = tuple(col_vec(off) for off in _OFFS)
    UNROLL = 10

    def repack(p):
        src = in_v[p]
        dst = out_v[p]

        def rbody(t, carry):
            r0 = t * UNROLL
            for u in range(UNROLL):
                rv = jnp.full((16,), r0 + u, jnp.int32)
                for off, cv in zip(_OFFS, colvs):
                    dst[r0 + u, pl.ds(off, 16)] = plsc.load_gather(
                        src, [rv, cv])
            return carry

        lax.fori_loop(0, CHUNK_ROWS // UNROLL, rbody, 0)

    start_in(0, 0)

    def pair_body(t, carry):
        c0 = 2 * t

        start_in(c0 + 1, 1)
        wait_in(c0, 0)

        @pl.when(t >= 1)
        def _():
            wait_out(c0 - 2, 0)

        repack(0)
        start_out(c0, 0)

        @pl.when(t + 1 < NPAIR)
        def _():
            start_in(c0 + 2, 0)

        wait_in(c0 + 1, 1)

        @pl.when(t >= 1)
        def _():
            wait_out(c0 - 1, 1)

        repack(1)
        start_out(c0 + 1, 1)
        return carry

    lax.fori_loop(0, NPAIR, pair_body, 0)
    wait_out(NCHUNK - 2, 0)
    wait_out(NCHUNK - 1, 1)


def kernel(observed_pose):
    return _sc_select(observed_pose)
